# baseline (speedup 1.0000x reference)
import functools

import jax
import jax.numpy as jnp
import numpy as np
from jax import lax
from jax.experimental import pallas as pl
from jax.experimental.pallas import tpu as pltpu
from jax.experimental.pallas import tpu_sc as plsc

_B = 16384
_A = 128

_LANES = 16
_CHUNK = 128


def _sample_body(n_workers, rows_per_worker, table_hbm, state_hbm, e_hbm,
                 out_hbm, idx_v, rows_v, e_v, out_v, sem):
    info = plsc.get_sparse_core_info()
    wid = lax.axis_index("s") * info.num_cores + lax.axis_index("c")
    base0 = wid * rows_per_worker
    n_chunks = rows_per_worker // _CHUNK

    def chunk_body(ci, carry):
        base = base0 + ci * _CHUNK
        pltpu.sync_copy(state_hbm.at[pl.ds(base, _CHUNK)], idx_v)
        gat = pltpu.async_copy(table_hbm.at[idx_v], rows_v, sem)
        pltpu.sync_copy(e_hbm.at[pl.ds(base, _CHUNK), :], e_v)
        gat.wait()

        lane = lax.iota(jnp.int32, _LANES)

        _BLK = 8

        def group_body(t, carry2):
            acc = jnp.zeros((_LANES,), jnp.int32)
            for blk in range(_LANES // _BLK):
                rs = [t * _LANES + blk * _BLK + i for i in range(_BLK)]
                mx = [jnp.full((_LANES,), -jnp.inf, jnp.float32)] * _BLK
                argj = [jnp.zeros((_LANES,), jnp.int32)] * _BLK
                for j in range(_A // _LANES):
                    for i in range(_BLK):
                        v = (rows_v[rs[i], pl.ds(j * _LANES, _LANES)]
                             * e_v[rs[i], pl.ds(j * _LANES, _LANES)])
                        upd = v > mx[i]
                        mx[i] = jnp.where(upd, v, mx[i])
                        argj[i] = jnp.where(upd, j, argj[i])
                a = [argj[i] * _LANES + lane for i in range(_BLK)]
                for s in (8, 4, 2, 1):
                    idx = lane ^ s
                    for i in range(_BLK):
                        pm = mx[i][idx]
                        pa = a[i][idx]
                        take = (pm > mx[i]) | ((pm == mx[i]) & (pa < a[i]))
                        mx[i] = jnp.where(take, pm, mx[i])
                        a[i] = jnp.where(take, pa, a[i])
                for i in range(_BLK):
                    acc = jnp.where(lane == blk * _BLK + i, a[i], acc)
            out_v[pl.ds(t * _LANES, _LANES)] = acc
            return carry2

        lax.fori_loop(0, _CHUNK // _LANES, group_body, 0)
        pltpu.sync_copy(out_v, out_hbm.at[pl.ds(base, _CHUNK)])
        return carry

    lax.fori_loop(0, n_chunks, chunk_body, 0)


@functools.cache
def _noise():
    with jax.ensure_compile_time_eval():
        g = jax.random.gumbel(jax.random.key(42), (_B, _A), jnp.float32)
        return np.asarray(jnp.exp(g))


@functools.cache
def _build():
    info = plsc.get_sparse_core_info()
    n_workers = info.num_cores * info.num_subcores
    rows_per_worker = _B // n_workers
    mesh = plsc.VectorSubcoreMesh(core_axis_name="c", subcore_axis_name="s")
    return pl.kernel(
        functools.partial(_sample_body, n_workers, rows_per_worker),
        mesh=mesh,
        out_type=jax.ShapeDtypeStruct((_B,), jnp.int32),
        scratch_types=[
            pltpu.VMEM((_CHUNK,), jnp.int32),
            pltpu.VMEM((_CHUNK, _A), jnp.float32),
            pltpu.VMEM((_CHUNK, _A), jnp.float32),
            pltpu.VMEM((_CHUNK,), jnp.int32),
            pltpu.SemaphoreType.DMA,
        ],
    )


def kernel(probs_a_s, state):
    return _build()(probs_a_s, state.astype(jnp.int32), _noise())


# speedup vs baseline: 1.1839x; 1.1839x over previous
import functools

import jax
import jax.numpy as jnp
import numpy as np
from jax import lax
from jax.experimental import pallas as pl
from jax.experimental.pallas import tpu as pltpu
from jax.experimental.pallas import tpu_sc as plsc

_B = 16384
_A = 128

_LANES = 16
_CHUNK = 128


def _sample_body(rows_per_worker, table_hbm, state_hbm, e_hbm, out_hbm,
                 idx_all, e_all, rows0, rows1, out_v, sem_e, sem0, sem1):
    info = plsc.get_sparse_core_info()
    wid = lax.axis_index("s") * info.num_cores + lax.axis_index("c")
    base0 = wid * rows_per_worker

    pltpu.sync_copy(state_hbm.at[pl.ds(base0, rows_per_worker)], idx_all)
    ecp = pltpu.async_copy(e_hbm.at[pl.ds(base0, rows_per_worker), :], e_all,
                           sem_e)
    pltpu.async_copy(table_hbm.at[idx_all.at[pl.ds(0, _CHUNK)]], rows0, sem0)
    pltpu.async_copy(table_hbm.at[idx_all.at[pl.ds(_CHUNK, _CHUNK)]], rows1,
                     sem1)
    ecp.wait()

    lane = lax.iota(jnp.int32, _LANES)

    def compute_chunk(rows_v, off, t, carry):
        acc = jnp.zeros((_LANES,), jnp.int32)
        for i in range(_LANES):
            r = t * _LANES + i
            mx = jnp.full((_LANES,), -jnp.inf, jnp.float32)
            argj = jnp.zeros((_LANES,), jnp.int32)
            for j in range(_A // _LANES):
                v = (rows_v[r, pl.ds(j * _LANES, _LANES)]
                     * e_all[off + r, pl.ds(j * _LANES, _LANES)])
                upd = v > mx
                mx = jnp.where(upd, v, mx)
                argj = jnp.where(upd, j, argj)
            a = argj * _LANES + lane
            for s in (8, 4, 2, 1):
                idx = lane ^ s
                pm = mx[idx]
                pa = a[idx]
                take = (pm > mx) | ((pm == mx) & (pa < a))
                mx = jnp.where(take, pm, mx)
                a = jnp.where(take, pa, a)
            acc = jnp.where(lane == i, a, acc)
        out_v[pl.ds(off + t * _LANES, _LANES)] = acc
        return carry

    n_groups = _CHUNK // _LANES

    def pair_body(pair, carry):
        off0 = pair * (2 * _CHUNK)
        pltpu.make_async_copy(
            table_hbm.at[idx_all.at[pl.ds(off0, _CHUNK)]], rows0, sem0).wait()
        lax.fori_loop(0, n_groups, functools.partial(compute_chunk, rows0,
                                                     off0), 0)

        @pl.when(pair == 0)
        def _():
            pltpu.async_copy(
                table_hbm.at[idx_all.at[pl.ds(2 * _CHUNK, _CHUNK)]], rows0,
                sem0)

        off1 = off0 + _CHUNK
        pltpu.make_async_copy(
            table_hbm.at[idx_all.at[pl.ds(off1, _CHUNK)]], rows1, sem1).wait()
        lax.fori_loop(0, n_groups, functools.partial(compute_chunk, rows1,
                                                     off1), 0)

        @pl.when(pair == 0)
        def _():
            pltpu.async_copy(
                table_hbm.at[idx_all.at[pl.ds(3 * _CHUNK, _CHUNK)]], rows1,
                sem1)

        return carry

    lax.fori_loop(0, rows_per_worker // (2 * _CHUNK), pair_body, 0)
    pltpu.sync_copy(out_v, out_hbm.at[pl.ds(base0, rows_per_worker)])


@functools.cache
def _noise():
    with jax.ensure_compile_time_eval():
        g = jax.random.gumbel(jax.random.key(42), (_B, _A), jnp.float32)
        return np.asarray(jnp.exp(g))


@functools.cache
def _build():
    info = plsc.get_sparse_core_info()
    n_workers = info.num_cores * info.num_subcores
    rows_per_worker = _B // n_workers
    mesh = plsc.VectorSubcoreMesh(core_axis_name="c", subcore_axis_name="s")
    return pl.kernel(
        functools.partial(_sample_body, rows_per_worker),
        mesh=mesh,
        out_type=jax.ShapeDtypeStruct((_B,), jnp.int32),
        scratch_types=[
            pltpu.VMEM((rows_per_worker,), jnp.int32),
            pltpu.VMEM((rows_per_worker, _A), jnp.float32),
            pltpu.VMEM((_CHUNK, _A), jnp.float32),
            pltpu.VMEM((_CHUNK, _A), jnp.float32),
            pltpu.VMEM((rows_per_worker,), jnp.int32),
            pltpu.SemaphoreType.DMA,
            pltpu.SemaphoreType.DMA,
            pltpu.SemaphoreType.DMA,
        ],
    )


def kernel(probs_a_s, state):
    return _build()(probs_a_s, state.astype(jnp.int32), _noise())
